# Initial kernel scaffold; baseline (speedup 1.0000x reference)
import jax
import jax.numpy as jnp
from jax.experimental import pallas as pl


def kernel(inputs, adapter_outputs, adapter_outputs_copy, residual, adapter_indices):
    raise NotImplementedError("write your pallas kernel here")


# speedup vs baseline: 4.0245x; 4.0245x over previous
import functools

import jax
import jax.numpy as jnp
from jax import lax
from jax.experimental import pallas as pl
from jax.experimental.pallas import tpu as pltpu
from jax.experimental.pallas import tpu_sc as plsc

NC, NS, L = 2, 16, 16
NW = NC * NS
BSZ, SEQ, NA, D = 2, 2048, 8, 1024
ROWS = BSZ * SEQ
RPW = ROWS // NW
CH = 32
NCHUNK = RPW // CH
WPB = NW // BSZ


def _body(ao, res, idxb, out, idx_v, idx_vv, gbuf, rbuf, sem):
    c = lax.axis_index("c")
    s = lax.axis_index("s")
    wid = s * NC + c
    b = wid // WPB
    base = wid * RPW

    pltpu.sync_copy(idxb.at[b], idx_vv)
    iv = idx_vv[...]

    for ch in range(NCHUNK):
        for j in range(CH // L):
            off = ch * CH + j * L
            idx_v[ch, pl.ds(j * L, L)] = (
                base + off + lax.iota(jnp.int32, L)) * NA + iv

    for ch in range(NCHUNK):
        r0 = base + ch * CH
        cp_r = pltpu.async_copy(res.at[pl.ds(r0, CH)], rbuf, sem)
        cp_g = pltpu.async_copy(ao.at[idx_v.at[ch]], gbuf, sem)
        cp_r.wait()
        cp_g.wait()

        def row_add(i, carry):
            for j in range(D // L):
                sl = pl.ds(j * L, L)
                rbuf[i, sl] = rbuf[i, sl] + gbuf[i, sl]
            return carry

        lax.fori_loop(0, CH, row_add, 0)
        pltpu.sync_copy(rbuf, out.at[pl.ds(r0, CH)])


_sc_call = functools.partial(
    pl.kernel,
    out_type=jax.ShapeDtypeStruct((ROWS, D), jnp.float32),
    mesh=plsc.VectorSubcoreMesh(core_axis_name="c", subcore_axis_name="s"),
    scratch_types=[
        pltpu.VMEM((NCHUNK, CH), jnp.int32),
        pltpu.VMEM((L,), jnp.int32),
        pltpu.VMEM((CH, D), jnp.float32),
        pltpu.VMEM((CH, D), jnp.float32),
        pltpu.SemaphoreType.DMA,
    ],
)(_body)


def kernel(inputs, adapter_outputs, adapter_outputs_copy, residual, adapter_indices):
    ao = adapter_outputs.reshape(ROWS * NA, D)
    res = residual.reshape(ROWS, D)
    idxb = jnp.broadcast_to(
        adapter_indices.astype(jnp.int32)[:, None], (BSZ, L))
    out = _sc_call(ao, res, idxb)
    return out.reshape(BSZ, SEQ, D)


# speedup vs baseline: 5.0355x; 1.2512x over previous
import functools

import jax
import jax.numpy as jnp
from jax import lax
from jax.experimental import pallas as pl
from jax.experimental.pallas import tpu as pltpu
from jax.experimental.pallas import tpu_sc as plsc

NC, NS, L = 2, 16, 16
NW = NC * NS
BSZ, SEQ, NA, D = 2, 2048, 8, 1024
ROWS = BSZ * SEQ
RPW = ROWS // NW
CH = 16
NCHUNK = RPW // CH
WPB = NW // BSZ


def _body(ao, res, idxb, out, idx_v, idx_vv,
          gbuf0, gbuf1, rbuf0, rbuf1,
          gsem0, gsem1, rsem0, rsem1, osem0, osem1):
    gbuf = (gbuf0, gbuf1)
    rbuf = (rbuf0, rbuf1)
    gsem = (gsem0, gsem1)
    rsem = (rsem0, rsem1)
    osem = (osem0, osem1)

    c = lax.axis_index("c")
    s = lax.axis_index("s")
    wid = s * NC + c
    b = wid // WPB
    base = wid * RPW

    pltpu.sync_copy(idxb.at[b], idx_vv)
    iv = idx_vv[...]

    for ch in range(NCHUNK):
        idx_v[ch, pl.ds(0, L)] = (
            base + ch * CH + lax.iota(jnp.int32, L)) * NA + iv

    def issue_in(ch):
        nb = ch % 2
        r0 = base + ch * CH
        g = pltpu.async_copy(ao.at[idx_v.at[ch]], gbuf[nb], gsem[nb])
        r = pltpu.async_copy(res.at[pl.ds(r0, CH)], rbuf[nb], rsem[nb])
        return g, r

    pend_in = {0: issue_in(0)}
    pend_out = {}
    for ch in range(NCHUNK):
        nb = ch % 2
        if ch + 1 < NCHUNK:
            if ch - 1 in pend_out:
                pend_out.pop(ch - 1).wait()
            pend_in[ch + 1] = issue_in(ch + 1)
        g, r = pend_in.pop(ch)
        g.wait()
        r.wait()

        def row_add(i, carry, _g=gbuf[nb], _r=rbuf[nb]):
            for j in range(D // L):
                sl = pl.ds(j * L, L)
                plsc.addupdate(_r.at[i, sl], _g[i, sl])
            return carry

        lax.fori_loop(0, CH, row_add, 0)
        pend_out[ch] = pltpu.async_copy(
            rbuf[nb], out.at[pl.ds(base + ch * CH, CH)], osem[nb])
    for cp in pend_out.values():
        cp.wait()


_sc_call = functools.partial(
    pl.kernel,
    out_type=jax.ShapeDtypeStruct((ROWS, D), jnp.float32),
    mesh=plsc.VectorSubcoreMesh(core_axis_name="c", subcore_axis_name="s"),
    scratch_types=[
        pltpu.VMEM((NCHUNK, CH), jnp.int32),
        pltpu.VMEM((L,), jnp.int32),
        pltpu.VMEM((CH, D), jnp.float32),
        pltpu.VMEM((CH, D), jnp.float32),
        pltpu.VMEM((CH, D), jnp.float32),
        pltpu.VMEM((CH, D), jnp.float32),
        pltpu.SemaphoreType.DMA,
        pltpu.SemaphoreType.DMA,
        pltpu.SemaphoreType.DMA,
        pltpu.SemaphoreType.DMA,
        pltpu.SemaphoreType.DMA,
        pltpu.SemaphoreType.DMA,
    ],
)(_body)


def kernel(inputs, adapter_outputs, adapter_outputs_copy, residual, adapter_indices):
    ao = adapter_outputs.reshape(ROWS * NA, D)
    res = residual.reshape(ROWS, D)
    idxb = jnp.broadcast_to(
        adapter_indices.astype(jnp.int32)[:, None], (BSZ, L))
    out = _sc_call(ao, res, idxb)
    return out.reshape(BSZ, SEQ, D)
